# baseline (speedup 1.0000x reference)
import functools

import jax
import jax.numpy as jnp
from jax import lax
from jax.experimental import pallas as pl
from jax.experimental.pallas import tpu as pltpu
from jax.experimental.pallas import tpu_sc as plsc

_B, _S, _D = 4, 2048, 1024
_NC, _NS = 2, 16
_NW = _NC * _NS
_RW = _S // _NW
_CH = 16
_NCH = _RW // _CH
_L = 16

_mesh = plsc.VectorSubcoreMesh(core_axis_name="c", subcore_axis_name="s")


@functools.partial(
    pl.kernel,
    mesh=_mesh,
    out_type=jax.ShapeDtypeStruct((_B * _S * _D,), jnp.float32),
    scratch_types=[
        pltpu.VMEM((_RW * _D,), jnp.float32),
        pltpu.VMEM((_CH * _D,), jnp.float32),
    ],
)
def _sc_add(x_hbm, pe_hbm, out_hbm, pe_v, x_v):
    wid = lax.axis_index("s") * _NC + lax.axis_index("c")
    base = wid * _RW * _D
    pltpu.sync_copy(pe_hbm.at[pl.ds(base, _RW * _D)], pe_v)
    for b in range(_B):
        for c in range(_NCH):
            off = b * _S * _D + base + c * _CH * _D
            pltpu.sync_copy(x_hbm.at[pl.ds(off, _CH * _D)], x_v)
            peo = c * _CH * _D

            def _add(j, carry, peo=peo):
                sl = pl.ds(j * _L, _L)
                x_v[sl] = x_v[sl] + pe_v[pl.ds(peo + j * _L, _L)]
                return carry

            lax.fori_loop(0, _CH * _D // _L, _add, 0)
            pltpu.sync_copy(x_v, out_hbm.at[pl.ds(off, _CH * _D)])


def kernel(x, pos_table):
    B, S, D = x.shape
    out = _sc_add(x.reshape(-1), pos_table[:S].reshape(-1))
    return out.reshape(B, S, D)


# speedup vs baseline: 1.7461x; 1.7461x over previous
import functools

import jax
import jax.numpy as jnp
from jax import lax
from jax.experimental import pallas as pl
from jax.experimental.pallas import tpu as pltpu
from jax.experimental.pallas import tpu_sc as plsc

_B, _S, _D = 4, 2048, 1024
_NC, _NS = 2, 16
_NW = _NC * _NS
_RW = _S // _NW
_CH = 16
_NCH = _RW // _CH
_NT = _B * _NCH
_L = 16
_NBUF = 3

_mesh = plsc.VectorSubcoreMesh(core_axis_name="c", subcore_axis_name="s")


@functools.partial(
    pl.kernel,
    mesh=_mesh,
    out_type=jax.ShapeDtypeStruct((_B * _S * _D,), jnp.float32),
    scratch_types=[
        pltpu.VMEM((_RW * _D,), jnp.float32),
        pltpu.VMEM((_CH * _D,), jnp.float32),
        pltpu.VMEM((_CH * _D,), jnp.float32),
        pltpu.VMEM((_CH * _D,), jnp.float32),
        pltpu.SemaphoreType.DMA,
        pltpu.SemaphoreType.DMA,
        pltpu.SemaphoreType.DMA,
        pltpu.SemaphoreType.DMA,
        pltpu.SemaphoreType.DMA,
        pltpu.SemaphoreType.DMA,
    ],
)
def _sc_add(x_hbm, pe_hbm, out_hbm, pe_v, b0, b1, b2, i0, i1, i2, o0, o1, o2):
    bufs = (b0, b1, b2)
    isems = (i0, i1, i2)
    osems = (o0, o1, o2)
    wid = lax.axis_index("s") * _NC + lax.axis_index("c")
    base = wid * _RW * _D
    pltpu.sync_copy(pe_hbm.at[pl.ds(base, _RW * _D)], pe_v)

    def off(t):
        b, c = divmod(t, _NCH)
        return b * _S * _D + base + c * _CH * _D

    in_d = {}
    out_d = {}
    for t in range(min(2, _NT)):
        bi = t % _NBUF
        in_d[t] = pltpu.async_copy(
            x_hbm.at[pl.ds(off(t), _CH * _D)], bufs[bi], isems[bi])
    for t in range(_NT):
        bi = t % _NBUF
        in_d[t].wait()
        peo = (t % _NCH) * _CH * _D
        buf = bufs[bi]

        @plsc.parallel_loop(0, _CH * _D // _L, 1, unroll=8)
        def _add(j, peo=peo, buf=buf):
            sl = pl.ds(j * _L, _L)
            buf[sl] = buf[sl] + pe_v[pl.ds(peo + j * _L, _L)]

        out_d[t] = pltpu.async_copy(
            buf, out_hbm.at[pl.ds(off(t), _CH * _D)], osems[bi])
        nt = t + 2
        if nt < _NT:
            if t >= 1:
                out_d[t - 1].wait()
            nbi = nt % _NBUF
            in_d[nt] = pltpu.async_copy(
                x_hbm.at[pl.ds(off(nt), _CH * _D)], bufs[nbi], isems[nbi])
    for t in range(max(0, _NT - 2), _NT):
        out_d[t].wait()


def kernel(x, pos_table):
    B, S, D = x.shape
    out = _sc_add(x.reshape(-1), pos_table[:S].reshape(-1))
    return out.reshape(B, S, D)


# speedup vs baseline: 4.2494x; 2.4337x over previous
import functools

import jax
import jax.numpy as jnp
from jax import lax
from jax.experimental import pallas as pl
from jax.experimental.pallas import tpu as pltpu
from jax.experimental.pallas import tpu_sc as plsc

_B, _S, _D = 4, 2048, 1024
_NC, _NS = 2, 16
_NW = _NC * _NS
_RW = _S // _NW
_CH = 16
_NCH = _RW // _CH
_NT = _B * _NCH
_L = 16
_CS = _D // _L
_NBUF = 3

_mesh = plsc.VectorSubcoreMesh(core_axis_name="c", subcore_axis_name="s")


@functools.partial(
    pl.kernel,
    mesh=_mesh,
    out_type=jax.ShapeDtypeStruct((_B, _S, _D), jnp.float32),
    scratch_types=[
        pltpu.VMEM((_RW, _D), jnp.float32),
        pltpu.VMEM((_CH, _D), jnp.float32),
        pltpu.VMEM((_CH, _D), jnp.float32),
        pltpu.VMEM((_CH, _D), jnp.float32),
        pltpu.SemaphoreType.DMA,
        pltpu.SemaphoreType.DMA,
        pltpu.SemaphoreType.DMA,
        pltpu.SemaphoreType.DMA,
        pltpu.SemaphoreType.DMA,
        pltpu.SemaphoreType.DMA,
    ],
    compiler_params=pltpu.CompilerParams(use_tc_tiling_on_sc=True),
)
def _sc_add(x_hbm, pe_hbm, out_hbm, pe_v, b0, b1, b2, i0, i1, i2, o0, o1, o2):
    bufs = (b0, b1, b2)
    isems = (i0, i1, i2)
    osems = (o0, o1, o2)
    wid = lax.axis_index("s") * _NC + lax.axis_index("c")
    base_r = wid * _RW
    pltpu.sync_copy(pe_hbm.at[pl.ds(base_r, _RW)], pe_v)

    def rows(t):
        b, c = divmod(t, _NCH)
        return b, c * _CH

    in_d = {}
    out_d = {}
    for t in range(min(2, _NT)):
        bi = t % _NBUF
        b, r0 = rows(t)
        in_d[t] = pltpu.async_copy(
            x_hbm.at[b, pl.ds(base_r + r0, _CH)], bufs[bi], isems[bi])
    for t in range(_NT):
        bi = t % _NBUF
        in_d[t].wait()
        b, r0 = rows(t)
        buf = bufs[bi]

        @plsc.parallel_loop(0, _CH * _CS, 1, unroll=8)
        def _add(j, r0=r0, buf=buf):
            r = j >> 6
            c = pl.multiple_of((j & (_CS - 1)) << 4, _L)
            sl = pl.ds(c, _L)
            buf[r, sl] = buf[r, sl] + pe_v[r0 + r, sl]

        out_d[t] = pltpu.async_copy(
            buf, out_hbm.at[b, pl.ds(base_r + r0, _CH)], osems[bi])
        nt = t + 2
        if nt < _NT:
            if t >= 1:
                out_d[t - 1].wait()
            nbi = nt % _NBUF
            nb, nr0 = rows(nt)
            in_d[nt] = pltpu.async_copy(
                x_hbm.at[nb, pl.ds(base_r + nr0, _CH)], bufs[nbi], isems[nbi])
    for t in range(max(0, _NT - 2), _NT):
        out_d[t].wait()


def kernel(x, pos_table):
    B, S, D = x.shape
    return _sc_add(x, pos_table[:S])


# speedup vs baseline: 7.7944x; 1.8342x over previous
import jax
import jax.numpy as jnp
from jax.experimental import pallas as pl


_BS = 512


def _body(x_ref, p_ref, o_ref):
    o_ref[...] = x_ref[...] + p_ref[...]


def kernel(x, pos_table):
    B, S, D = x.shape
    pe = pos_table[:S]
    grid = (S // _BS, B)
    return pl.pallas_call(
        _body,
        grid=grid,
        in_specs=[
            pl.BlockSpec((1, _BS, D), lambda s, b: (b, s, 0)),
            pl.BlockSpec((_BS, D), lambda s, b: (s, 0)),
        ],
        out_specs=pl.BlockSpec((1, _BS, D), lambda s, b: (b, s, 0)),
        out_shape=jax.ShapeDtypeStruct((B, S, D), x.dtype),
    )(x, pe)


# speedup vs baseline: 9.3044x; 1.1937x over previous
import jax
import jax.numpy as jnp
from jax.experimental import pallas as pl


_BS = 2048


def _body(x_ref, p_ref, o_ref):
    o_ref[...] = x_ref[...] + p_ref[...]


def kernel(x, pos_table):
    B, S, D = x.shape
    pe = pos_table[:S]
    grid = (S // _BS, B)
    return pl.pallas_call(
        _body,
        grid=grid,
        in_specs=[
            pl.BlockSpec((1, _BS, D), lambda s, b: (b, s, 0)),
            pl.BlockSpec((_BS, D), lambda s, b: (s, 0)),
        ],
        out_specs=pl.BlockSpec((1, _BS, D), lambda s, b: (b, s, 0)),
        out_shape=jax.ShapeDtypeStruct((B, S, D), x.dtype),
    )(x, pe)
